# Initial kernel scaffold; baseline (speedup 1.0000x reference)
import jax
import jax.numpy as jnp
from jax.experimental import pallas as pl


def kernel(x, edge_index, W, b):
    raise NotImplementedError("write your pallas kernel here")


# speedup vs baseline: 27.1502x; 27.1502x over previous
import functools

import jax
import jax.numpy as jnp
from jax import lax
from jax.experimental import pallas as pl
from jax.experimental.pallas import tpu as pltpu
from jax.experimental.pallas import tpu_sc as plsc

N = 10000
E = 320000
F_IN = 128
F_OUT = 16

NC = 2
NS = 16
NW = NC * NS
N_PAD = 10240
RPT = N_PAD // NS
C = 128
EPT = ((E + NW * C - 1) // (NW * C)) * C
E_PAD = EPT * NW
NCHUNK = EPT // C

_MESH = plsc.VectorSubcoreMesh(core_axis_name="c", subcore_axis_name="s")
_SC_PARAMS = pltpu.CompilerParams(use_tc_tiling_on_sc=False)


@functools.partial(
    pl.kernel,
    out_type=jax.ShapeDtypeStruct((NC, N_PAD, F_OUT), jnp.float32),
    mesh=_MESH,
    scratch_types=[
        pltpu.VMEM((C,), jnp.int32),
        pltpu.VMEM((C, F_OUT), jnp.float32),
        pltpu.VMEM((RPT, F_OUT), jnp.float32),
        pltpu.VMEM_SHARED((N_PAD, F_OUT), jnp.float32),
        pltpu.SemaphoreType.DMA,
    ],
    compiler_params=_SC_PARAMS,
)
def _deg_kernel(col_hbm, onehot_hbm, zeros_hbm, out_hbm,
                cidx, oh, stage, acc_sp, sem):
    cid = lax.axis_index("c")
    sid = lax.axis_index("s")
    wid = cid * NS + sid
    tbase = sid * RPT
    pltpu.sync_copy(zeros_hbm, stage)
    pltpu.sync_copy(stage, acc_sp.at[pl.ds(tbase, RPT)])
    pltpu.sync_copy(onehot_hbm, oh)
    plsc.subcore_barrier()

    ebase = wid * EPT

    def body(i, carry):
        off = pl.multiple_of(ebase + i * C, C)
        pltpu.sync_copy(col_hbm.at[pl.ds(off, C)], cidx)
        pltpu.sync_copy(oh, acc_sp.at[cidx], add=True)
        return carry

    lax.fori_loop(0, NCHUNK, body, 0)
    plsc.subcore_barrier()
    pltpu.sync_copy(acc_sp.at[pl.ds(tbase, RPT)], stage)
    pltpu.sync_copy(stage, out_hbm.at[cid, pl.ds(tbase, RPT)])


@functools.partial(
    pl.kernel,
    out_type=jax.ShapeDtypeStruct((NC, N_PAD, F_OUT), jnp.float32),
    mesh=_MESH,
    scratch_types=[
        pltpu.VMEM((C,), jnp.int32),
        pltpu.VMEM((C,), jnp.int32),
        pltpu.VMEM((C, F_OUT), jnp.float32),
        pltpu.VMEM((RPT, F_OUT), jnp.float32),
        pltpu.VMEM_SHARED((N_PAD, F_OUT), jnp.float32),
        pltpu.SemaphoreType.DMA,
    ],
    compiler_params=_SC_PARAMS,
)
def _prop_kernel(u_hbm, row_hbm, col_hbm, zeros_hbm, out_hbm,
                 ridx, cidx, rows, stage, acc_sp, sem):
    cid = lax.axis_index("c")
    sid = lax.axis_index("s")
    wid = cid * NS + sid
    tbase = sid * RPT
    pltpu.sync_copy(zeros_hbm, stage)
    pltpu.sync_copy(stage, acc_sp.at[pl.ds(tbase, RPT)])
    plsc.subcore_barrier()

    ebase = wid * EPT

    def body(i, carry):
        off = pl.multiple_of(ebase + i * C, C)
        pltpu.sync_copy(row_hbm.at[pl.ds(off, C)], ridx)
        pltpu.sync_copy(col_hbm.at[pl.ds(off, C)], cidx)
        pltpu.async_copy(u_hbm.at[ridx], rows, sem).wait()
        pltpu.sync_copy(rows, acc_sp.at[cidx], add=True)
        return carry

    lax.fori_loop(0, NCHUNK, body, 0)
    plsc.subcore_barrier()
    pltpu.sync_copy(acc_sp.at[pl.ds(tbase, RPT)], stage)
    pltpu.sync_copy(stage, out_hbm.at[cid, pl.ds(tbase, RPT)])


def _scale_body(x_ref, wt_ref, degp_ref, u_ref, disb_ref):
    y = jnp.dot(x_ref[...], wt_ref[...], preferred_element_type=jnp.float32)
    deg = degp_ref[0, :, 0:1] + degp_ref[1, :, 0:1] + 1.0
    dis = lax.rsqrt(deg)
    u_ref[...] = dis * y
    disb_ref[...] = jnp.broadcast_to(dis, (N_PAD, F_OUT))


_scale_call = pl.pallas_call(
    _scale_body,
    out_shape=(
        jax.ShapeDtypeStruct((N_PAD, F_OUT), jnp.float32),
        jax.ShapeDtypeStruct((N_PAD, F_OUT), jnp.float32),
    ),
)


def _finish_body(accp_ref, u_ref, disb_ref, b_ref, o_ref):
    acc = accp_ref[0] + accp_ref[1] + u_ref[...]
    h = disb_ref[...] * acc + b_ref[...]
    m = jnp.max(h, axis=1, keepdims=True)
    e = jnp.exp(h - m)
    s = jnp.sum(e, axis=1, keepdims=True)
    o_ref[...] = h - m - jnp.log(s)


_finish_call = pl.pallas_call(
    _finish_body,
    out_shape=jax.ShapeDtypeStruct((N_PAD, F_OUT), jnp.float32),
)


def kernel(x, edge_index, W, b):
    row = edge_index[0]
    col = edge_index[1]
    npad_edges = E_PAD - E
    pad_ids = (N + jnp.arange(npad_edges, dtype=jnp.int32) % (N_PAD - N))
    row_p = jnp.concatenate([row, pad_ids])
    col_p = jnp.concatenate([col, pad_ids])

    onehot = jnp.zeros((C, F_OUT), jnp.float32).at[:, 0].set(1.0)
    zeros = jnp.zeros((RPT, F_OUT), jnp.float32)

    deg_parts = _deg_kernel(col_p, onehot, zeros)
    x_pad = jnp.pad(x, ((0, N_PAD - N), (0, 0)))
    u, dis_b = _scale_call(x_pad, W.T, deg_parts)
    acc_parts = _prop_kernel(u, row_p, col_p, zeros)
    out = _finish_call(acc_parts, u, dis_b, b.reshape(1, F_OUT))
    return out[:N]


# speedup vs baseline: 59.1878x; 2.1800x over previous
import functools

import jax
import jax.numpy as jnp
from jax import lax
from jax.experimental import pallas as pl
from jax.experimental.pallas import tpu as pltpu
from jax.experimental.pallas import tpu_sc as plsc

N = 10000
E = 320000
F_IN = 128
F_OUT = 16

NC = 2
NS = 16
NW = NC * NS
N_PAD = 10240
RPT = N_PAD // NS
C = 128
NBUF = 4
NCHUNK = 80
EPT = NCHUNK * C
E_PAD = EPT * NW

_MESH = plsc.VectorSubcoreMesh(core_axis_name="c", subcore_axis_name="s")
_SC_PARAMS = pltpu.CompilerParams(use_tc_tiling_on_sc=False)


@functools.partial(
    pl.kernel,
    out_type=jax.ShapeDtypeStruct((NC, N_PAD, F_OUT), jnp.float32),
    mesh=_MESH,
    scratch_types=[
        pltpu.VMEM((NCHUNK, C), jnp.int32),
        pltpu.VMEM((C, F_OUT), jnp.float32),
        pltpu.VMEM((RPT, F_OUT), jnp.float32),
        pltpu.VMEM_SHARED((N_PAD, F_OUT), jnp.float32),
        pltpu.SemaphoreType.DMA,
    ],
    compiler_params=_SC_PARAMS,
)
def _deg_kernel(col_hbm, onehot_hbm, zeros_hbm, out_hbm,
                cidx, oh, stage, acc_sp, sem):
    cid = lax.axis_index("c")
    sid = lax.axis_index("s")
    wid = cid * NS + sid
    tbase = sid * RPT
    pltpu.sync_copy(zeros_hbm, stage)
    pltpu.sync_copy(stage, acc_sp.at[pl.ds(tbase, RPT)])
    pltpu.sync_copy(onehot_hbm, oh)
    pltpu.sync_copy(col_hbm.at[wid], cidx)
    plsc.subcore_barrier()

    def body(i, carry):
        pltpu.sync_copy(oh, acc_sp.at[cidx.at[i]], add=True)
        return carry

    lax.fori_loop(0, NCHUNK, body, 0)
    plsc.subcore_barrier()
    pltpu.sync_copy(acc_sp.at[pl.ds(tbase, RPT)], stage)
    pltpu.sync_copy(stage, out_hbm.at[cid, pl.ds(tbase, RPT)])


@functools.partial(
    pl.kernel,
    out_type=jax.ShapeDtypeStruct((NC, N_PAD, F_OUT), jnp.float32),
    mesh=_MESH,
    scratch_types=[
        pltpu.VMEM((NCHUNK, C), jnp.int32),
        pltpu.VMEM((NCHUNK, C), jnp.int32),
        pltpu.VMEM((NBUF, C, F_OUT), jnp.float32),
        pltpu.VMEM((RPT, F_OUT), jnp.float32),
        pltpu.VMEM_SHARED((N_PAD, F_OUT), jnp.float32),
        pltpu.SemaphoreType.DMA((NBUF,)),
        pltpu.SemaphoreType.DMA,
    ],
    compiler_params=_SC_PARAMS,
)
def _prop_kernel(u_hbm, row_hbm, col_hbm, zeros_hbm, out_hbm,
                 ridx, cidx, rows, stage, acc_sp, gsem, sem):
    cid = lax.axis_index("c")
    sid = lax.axis_index("s")
    wid = cid * NS + sid
    tbase = sid * RPT
    pltpu.sync_copy(zeros_hbm, stage)
    pltpu.sync_copy(stage, acc_sp.at[pl.ds(tbase, RPT)])
    pltpu.sync_copy(row_hbm.at[wid], ridx)
    pltpu.sync_copy(col_hbm.at[wid], cidx)
    plsc.subcore_barrier()

    for b in range(NBUF):
        pltpu.async_copy(u_hbm.at[ridx.at[b]], rows.at[b], gsem.at[b])

    def body(i0, carry):
        for b in range(NBUF):
            i = i0 * NBUF + b
            pltpu.make_async_copy(u_hbm.at[ridx.at[i]], rows.at[b],
                                  gsem.at[b]).wait()
            pltpu.sync_copy(rows.at[b], acc_sp.at[cidx.at[i]], add=True)

            @pl.when(i + NBUF < NCHUNK)
            def _():
                pltpu.async_copy(u_hbm.at[ridx.at[i + NBUF]], rows.at[b],
                                 gsem.at[b])
        return carry

    lax.fori_loop(0, NCHUNK // NBUF, body, 0)
    plsc.subcore_barrier()
    pltpu.sync_copy(acc_sp.at[pl.ds(tbase, RPT)], stage)
    pltpu.sync_copy(stage, out_hbm.at[cid, pl.ds(tbase, RPT)])


def _scale_body(x_ref, wt_ref, degp_ref, u_ref, disb_ref):
    y = jnp.dot(x_ref[...], wt_ref[...], preferred_element_type=jnp.float32)
    deg = degp_ref[0, :, 0:1] + degp_ref[1, :, 0:1] + 1.0
    dis = lax.rsqrt(deg)
    u_ref[...] = dis * y
    disb_ref[...] = jnp.broadcast_to(dis, (N_PAD, F_OUT))


_scale_call = pl.pallas_call(
    _scale_body,
    out_shape=(
        jax.ShapeDtypeStruct((N_PAD, F_OUT), jnp.float32),
        jax.ShapeDtypeStruct((N_PAD, F_OUT), jnp.float32),
    ),
)


def _finish_body(accp_ref, u_ref, disb_ref, b_ref, o_ref):
    acc = accp_ref[0] + accp_ref[1] + u_ref[...]
    h = disb_ref[...] * acc + b_ref[...]
    m = jnp.max(h, axis=1, keepdims=True)
    e = jnp.exp(h - m)
    s = jnp.sum(e, axis=1, keepdims=True)
    o_ref[...] = h - m - jnp.log(s)


_finish_call = pl.pallas_call(
    _finish_body,
    out_shape=jax.ShapeDtypeStruct((N_PAD, F_OUT), jnp.float32),
)


def kernel(x, edge_index, W, b):
    row = edge_index[0]
    col = edge_index[1]
    npad_edges = E_PAD - E
    pad_ids = (N + jnp.arange(npad_edges, dtype=jnp.int32) % (N_PAD - N))
    row_r = jnp.concatenate([row, pad_ids]).reshape(NW, NCHUNK, C)
    col_r = jnp.concatenate([col, pad_ids]).reshape(NW, NCHUNK, C)

    onehot = jnp.zeros((C, F_OUT), jnp.float32).at[:, 0].set(1.0)
    zeros = jnp.zeros((RPT, F_OUT), jnp.float32)

    deg_parts = _deg_kernel(col_r, onehot, zeros)
    x_pad = jnp.pad(x, ((0, N_PAD - N), (0, 0)))
    u, dis_b = _scale_call(x_pad, W.T, deg_parts)
    acc_parts = _prop_kernel(u, row_r, col_r, zeros)
    out = _finish_call(acc_parts, u, dis_b, b.reshape(1, F_OUT))
    return out[:N]
